# baseline (speedup 1.0000x reference)
import functools

import jax
import jax.numpy as jnp
from jax import lax
from jax.experimental import pallas as pl
from jax.experimental.pallas import tpu as pltpu
from jax.experimental.pallas import tpu_sc as plsc

D_MODEL = 1024
BATCH = 4
SEQ_LEN = 4096
NUM_ROWS = 16384
NB = 2
CHUNK = 32
LANES = 16
NGRP = D_MODEL // LANES


@jax.jit
def _sc_embed(idx2, tbl_t):
    info = plsc.get_sparse_core_info()
    nc, ns = info.num_cores, info.num_subcores
    nw = nc * ns
    per_w = NUM_ROWS // nw
    n_chunks = per_w // CHUNK
    assert idx2.shape == (nw, per_w)
    assert tbl_t.shape == (3 * D_MODEL,)

    mesh = plsc.VectorSubcoreMesh(core_axis_name="c", subcore_axis_name="s")

    @functools.partial(
        pl.kernel,
        mesh=mesh,
        out_type=jax.ShapeDtypeStruct((BATCH, SEQ_LEN, D_MODEL), jnp.float32),
        scratch_types=(
            [pltpu.VMEM((per_w,), jnp.int32),
             pltpu.VMEM((3 * D_MODEL,), jnp.float32),
             pltpu.VMEM((CHUNK * LANES,), jnp.int32)]
            + [pltpu.VMEM((CHUNK, D_MODEL), jnp.float32) for _ in range(NB)]
            + [pltpu.SemaphoreType.DMA for _ in range(NB)]
        ),
    )
    def k(idx_hbm, tbl_hbm, out_hbm, idx_v, tbl_v, bidx_v, *rest):
        bufs = rest[:NB]
        ssems = rest[NB:]
        wid = lax.axis_index("s") * nc + lax.axis_index("c")
        wpb = SEQ_LEN // per_w
        bi = wid // wpb
        sbase = (wid % wpb) * per_w
        pltpu.sync_copy(idx_hbm.at[wid], idx_v)
        pltpu.sync_copy(tbl_hbm, tbl_v)

        GC = 8

        def fill_chunk(g, buf, bidx_v):
            for gr in range(CHUNK // LANES):
                idx16 = idx_v[pl.ds(g * CHUNK + gr * LANES, LANES)]
                for r in range(LANES):
                    bidx_v[pl.ds((gr * LANES + r) * LANES, LANES)] = (
                        jnp.full((LANES,), idx16[r], jnp.int32))
            for ct in range(NGRP // GC):
                tregs = [
                    [tbl_v[pl.ds(s * D_MODEL + (ct * GC + j) * LANES,
                                 LANES)] for s in range(3)]
                    for j in range(GC)
                ]

                def row_body(r):
                    bidx = bidx_v[pl.ds(r * LANES, LANES)]
                    m0 = bidx == 0
                    m1 = bidx == 1
                    for j in range(GC):
                        t0, t1, t2 = tregs[j]
                        val = jnp.where(m0, t0, jnp.where(m1, t1, t2))
                        buf[r, pl.ds((ct * GC + j) * LANES, LANES)] = val

                plsc.parallel_loop(0, CHUNK, unroll=2)(row_body)

        csz = CHUNK * D_MODEL

        def outer_body(t, _):
            for b in range(NB):
                g = t * NB + b

                @pl.when(t > 0)
                def _wait_prev():
                    pltpu.make_async_copy(
                        bufs[b], out_hbm.at[bi, pl.ds(sbase, CHUNK)],
                        ssems[b]).wait()

                fill_chunk(g, bufs[b], bidx_v)
                pltpu.async_copy(
                    bufs[b],
                    out_hbm.at[bi, pl.ds(sbase + g * CHUNK, CHUNK)],
                    ssems[b])
            return 0

        lax.fori_loop(0, n_chunks // NB, outer_body, 0)
        for b in range(NB):
            pltpu.make_async_copy(
                bufs[b], out_hbm.at[bi, pl.ds(sbase, CHUNK)],
                ssems[b]).wait()

    return k(idx2, tbl_t)


def kernel(segment_input, table):
    info = plsc.get_sparse_core_info()
    nw = info.num_cores * info.num_subcores
    per_w = NUM_ROWS // nw
    idx2 = segment_input.astype(jnp.int32).reshape(nw, per_w)
    return _sc_embed(idx2, table.reshape(-1))


# speedup vs baseline: 1.0081x; 1.0081x over previous
import functools

import jax
import jax.numpy as jnp
from jax import lax
from jax.experimental import pallas as pl
from jax.experimental.pallas import tpu as pltpu
from jax.experimental.pallas import tpu_sc as plsc

D_MODEL = 1024
BATCH = 4
SEQ_LEN = 4096
NUM_ROWS = 16384
NB = 2
CHUNK = 32
LANES = 16
NGRP = D_MODEL // LANES


@jax.jit
def _sc_embed(idx2, tbl_t):
    info = plsc.get_sparse_core_info()
    nc, ns = info.num_cores, info.num_subcores
    nw = nc * ns
    per_w = NUM_ROWS // nw
    n_chunks = per_w // CHUNK
    assert idx2.shape == (nw, per_w)
    assert tbl_t.shape == (nw, 3 * D_MODEL)

    mesh = plsc.VectorSubcoreMesh(core_axis_name="c", subcore_axis_name="s")

    @functools.partial(
        pl.kernel,
        mesh=mesh,
        out_type=jax.ShapeDtypeStruct((BATCH, SEQ_LEN, D_MODEL), jnp.float32),
        scratch_types=(
            [pltpu.VMEM((per_w,), jnp.int32),
             pltpu.VMEM((3 * D_MODEL,), jnp.float32),
             pltpu.VMEM((CHUNK * LANES,), jnp.int32)]
            + [pltpu.VMEM((CHUNK, D_MODEL), jnp.float32) for _ in range(NB)]
            + [pltpu.SemaphoreType.DMA for _ in range(NB)]
        ),
    )
    def k(idx_hbm, tbl_hbm, out_hbm, idx_v, tbl_v, bidx_v, *rest):
        bufs = rest[:NB]
        ssems = rest[NB:]
        wid = lax.axis_index("s") * nc + lax.axis_index("c")
        wpb = SEQ_LEN // per_w
        bi = wid // wpb
        sbase = (wid % wpb) * per_w
        pltpu.sync_copy(idx_hbm.at[wid], idx_v)
        pltpu.sync_copy(tbl_hbm.at[wid], tbl_v)

        GC = 8

        def fill_chunk(g, buf, bidx_v):
            for gr in range(CHUNK // LANES):
                idx16 = idx_v[pl.ds(g * CHUNK + gr * LANES, LANES)]
                for r in range(LANES):
                    bidx_v[pl.ds((gr * LANES + r) * LANES, LANES)] = (
                        jnp.full((LANES,), idx16[r], jnp.int32))
            for ct in range(NGRP // GC):
                tregs = [
                    [tbl_v[pl.ds(s * D_MODEL + (ct * GC + j) * LANES,
                                 LANES)] for s in range(3)]
                    for j in range(GC)
                ]

                def row_body(r):
                    bidx = bidx_v[pl.ds(r * LANES, LANES)]
                    m0 = bidx == 0
                    m1 = bidx == 1
                    for j in range(GC):
                        t0, t1, t2 = tregs[j]
                        val = jnp.where(m0, t0, jnp.where(m1, t1, t2))
                        buf[r, pl.ds((ct * GC + j) * LANES, LANES)] = val

                plsc.parallel_loop(0, CHUNK, unroll=2)(row_body)

        csz = CHUNK * D_MODEL

        def outer_body(t, _):
            for b in range(NB):
                g = t * NB + b

                @pl.when(t > 0)
                def _wait_prev():
                    pltpu.make_async_copy(
                        bufs[b], out_hbm.at[bi, pl.ds(sbase, CHUNK)],
                        ssems[b]).wait()

                fill_chunk(g, bufs[b], bidx_v)
                pltpu.async_copy(
                    bufs[b],
                    out_hbm.at[bi, pl.ds(sbase + g * CHUNK, CHUNK)],
                    ssems[b])
            return 0

        lax.fori_loop(0, n_chunks // NB, outer_body, 0)
        for b in range(NB):
            pltpu.make_async_copy(
                bufs[b], out_hbm.at[bi, pl.ds(sbase, CHUNK)],
                ssems[b]).wait()

    return k(idx2, tbl_t)


def kernel(segment_input, table):
    info = plsc.get_sparse_core_info()
    nw = info.num_cores * info.num_subcores
    per_w = NUM_ROWS // nw
    idx2 = segment_input.astype(jnp.int32).reshape(nw, per_w)
    tbl_r = jnp.tile(table.reshape(1, -1), (nw, 1))
    return _sc_embed(idx2, tbl_r)


# speedup vs baseline: 1.1006x; 1.0917x over previous
import functools

import jax
import jax.numpy as jnp
from jax import lax
from jax.experimental import pallas as pl
from jax.experimental.pallas import tpu as pltpu
from jax.experimental.pallas import tpu_sc as plsc

D_MODEL = 1024
BATCH = 4
SEQ_LEN = 4096
NUM_ROWS = 16384


@jax.jit
def _sc_embed(idx2, tbl_r):
    info = plsc.get_sparse_core_info()
    nc, ns = info.num_cores, info.num_subcores
    nw = nc * ns
    per_w = NUM_ROWS // nw
    assert idx2.shape == (nw, per_w)
    assert tbl_r.shape == (nw, 3 * D_MODEL)

    mesh = plsc.VectorSubcoreMesh(core_axis_name="c", subcore_axis_name="s")

    @functools.partial(
        pl.kernel,
        mesh=mesh,
        out_type=jax.ShapeDtypeStruct((BATCH, SEQ_LEN, D_MODEL), jnp.float32),
        scratch_types=(
            pltpu.VMEM((per_w,), jnp.int32),
            pltpu.VMEM((3 * D_MODEL,), jnp.float32),
            pltpu.SemaphoreType.DMA,
        ),
    )
    def k(idx_hbm, tbl_hbm, out_hbm, idx_v, tbl_v, sem):
        wid = lax.axis_index("s") * nc + lax.axis_index("c")
        wpb = SEQ_LEN // per_w
        bi = wid // wpb
        sbase = (wid % wpb) * per_w
        pltpu.sync_copy(idx_hbm.at[wid], idx_v)
        pltpu.sync_copy(tbl_hbm.at[wid], tbl_v)

        def issue(i, _):
            s = idx_v[pl.ds(i, 1)][0]
            pltpu.async_copy(
                tbl_v.at[pl.ds(s * D_MODEL, D_MODEL)],
                out_hbm.at[bi, sbase + i],
                sem)
            return 0

        lax.fori_loop(0, per_w, issue, 0)

        def drain(i, _):
            pltpu.make_async_copy(
                tbl_v.at[pl.ds(0, D_MODEL)],
                out_hbm.at[bi, sbase],
                sem).wait()
            return 0

        lax.fori_loop(0, per_w, drain, 0)

    return k(idx2, tbl_r)


def kernel(segment_input, table):
    info = plsc.get_sparse_core_info()
    nw = info.num_cores * info.num_subcores
    per_w = NUM_ROWS // nw
    idx2 = segment_input.astype(jnp.int32).reshape(nw, per_w)
    tbl_r = jnp.tile(table.reshape(1, -1), (nw, 1))
    return _sc_embed(idx2, tbl_r)
